# Initial kernel scaffold; baseline (speedup 1.0000x reference)
import jax
import jax.numpy as jnp
from jax.experimental import pallas as pl


def kernel(x, token_table, pos_table):
    raise NotImplementedError("write your pallas kernel here")


# speedup vs baseline: 1.2328x; 1.2328x over previous
import functools

import jax
import jax.numpy as jnp
from jax import lax
from jax.experimental import pallas as pl
from jax.experimental.pallas import tpu as pltpu
from jax.experimental.pallas import tpu_sc as plsc

B = 4096
T = 200
D = 32
FLAT = B * T
CHUNK = 100
PAIR = 2 * CHUNK


@functools.lru_cache(maxsize=None)
def _sc_kernel():
    info = plsc.get_sparse_core_info()
    NC, NS = info.num_cores, info.num_subcores
    NW = NC * NS
    rows_per_w = FLAT // NW
    pairs = rows_per_w // PAIR
    idx_rows = rows_per_w // CHUNK

    mesh = plsc.VectorSubcoreMesh(core_axis_name="c", subcore_axis_name="s")

    @functools.partial(
        pl.kernel,
        mesh=mesh,
        compiler_params=pltpu.CompilerParams(use_tc_tiling_on_sc=False),
        out_type=jax.ShapeDtypeStruct((FLAT, D), jnp.float32),
        scratch_types=[
            pltpu.VMEM((idx_rows, CHUNK), jnp.int32),
            pltpu.VMEM((T, D), jnp.float32),
            pltpu.VMEM((PAIR, D), jnp.float32),
            pltpu.SemaphoreType.DMA,
        ],
    )
    def k(x_hbm, tok_hbm, pos_hbm, out_hbm, idx_v, pos_v, buf_v, sem):
        cid = lax.axis_index("c")
        sid = lax.axis_index("s")
        wid = sid * NC + cid
        base = wid * rows_per_w

        pltpu.sync_copy(x_hbm.at[pl.ds(wid * idx_rows, idx_rows), :], idx_v)
        pltpu.sync_copy(pos_hbm, pos_v)

        def pair_body(p, _):
            g0 = pltpu.async_copy(
                tok_hbm.at[idx_v.at[2 * p]], buf_v.at[pl.ds(0, CHUNK), :], sem)
            g1 = pltpu.async_copy(
                tok_hbm.at[idx_v.at[2 * p + 1]], buf_v.at[pl.ds(CHUNK, CHUNK), :], sem)
            g0.wait()
            g1.wait()

            def add_row(r, _):
                buf_v[r, pl.ds(0, 16)] = buf_v[r, pl.ds(0, 16)] + pos_v[r, pl.ds(0, 16)]
                buf_v[r, pl.ds(16, 16)] = buf_v[r, pl.ds(16, 16)] + pos_v[r, pl.ds(16, 16)]
                return 0

            lax.fori_loop(0, T, add_row, 0)
            pltpu.sync_copy(buf_v, out_hbm.at[pl.ds(base + p * PAIR, PAIR), :])
            return 0

        lax.fori_loop(0, pairs, pair_body, 0)

    return k


def kernel(x, token_table, pos_table):
    x2 = x.reshape(FLAT // CHUNK, CHUNK).astype(jnp.int32)
    out = _sc_kernel()(x2, token_table, pos_table)
    return out.reshape(B, T, D)


# speedup vs baseline: 1.4905x; 1.2090x over previous
import functools

import jax
import jax.numpy as jnp
from jax import lax
from jax.experimental import pallas as pl
from jax.experimental.pallas import tpu as pltpu
from jax.experimental.pallas import tpu_sc as plsc

B = 4096
T = 200
D = 32
FLAT = B * T
CHUNK = 100
PAIR = 2 * CHUNK
NBUF = 4
ROW_UNROLL = 8


@functools.lru_cache(maxsize=None)
def _sc_kernel():
    info = plsc.get_sparse_core_info()
    NC, NS = info.num_cores, info.num_subcores
    NW = NC * NS
    rows_per_w = FLAT // NW
    pairs = rows_per_w // PAIR
    steps = pairs // NBUF
    idx_rows = rows_per_w // CHUNK

    mesh = plsc.VectorSubcoreMesh(core_axis_name="c", subcore_axis_name="s")

    @functools.partial(
        pl.kernel,
        mesh=mesh,
        compiler_params=pltpu.CompilerParams(use_tc_tiling_on_sc=False),
        out_type=jax.ShapeDtypeStruct((FLAT, D), jnp.float32),
        scratch_types=[
            pltpu.VMEM((idx_rows, CHUNK), jnp.int32),
            pltpu.VMEM((T, D), jnp.float32),
            pltpu.VMEM((NBUF, PAIR, D), jnp.float32),
            pltpu.VMEM((NBUF, PAIR, D), jnp.float32),
            *([pltpu.SemaphoreType.DMA] * (2 * NBUF)),
        ],
    )
    def k(x_hbm, tok_hbm, pos_hbm, out_hbm, idx_v, pos_v, gbuf, obuf, *sems):
        gsem = sems[:NBUF]
        osem = sems[NBUF:]
        cid = lax.axis_index("c")
        sid = lax.axis_index("s")
        wid = sid * NC + cid
        base = wid * rows_per_w

        pltpu.sync_copy(x_hbm.at[pl.ds(wid * idx_rows, idx_rows), :], idx_v)
        pltpu.sync_copy(pos_hbm, pos_v)

        def issue_gather(p, b):
            pltpu.async_copy(
                tok_hbm.at[idx_v.at[2 * p]],
                gbuf.at[b, pl.ds(0, CHUNK), :], gsem[b])
            pltpu.async_copy(
                tok_hbm.at[idx_v.at[2 * p + 1]],
                gbuf.at[b, pl.ds(CHUNK, CHUNK), :], gsem[b])

        def wait_gather(b):
            for half in (0, CHUNK):
                pltpu.make_async_copy(
                    tok_hbm.at[idx_v.at[0]],
                    gbuf.at[b, pl.ds(half, CHUNK), :], gsem[b]).wait()

        def issue_out(p, b):
            pltpu.async_copy(
                obuf.at[b], out_hbm.at[pl.ds(base + p * PAIR, PAIR), :], osem[b])

        def wait_out(b):
            pltpu.make_async_copy(
                obuf.at[b], out_hbm.at[pl.ds(base, PAIR), :], osem[b]).wait()

        def add_pos(b):
            def add_body(i, _):
                r0 = i * ROW_UNROLL
                for dr in range(ROW_UNROLL):
                    r = r0 + dr
                    for h in (0, 16):
                        obuf[b, r, pl.ds(h, 16)] = (
                            gbuf[b, r, pl.ds(h, 16)] + pos_v[r, pl.ds(h, 16)])
                return 0
            lax.fori_loop(0, T // ROW_UNROLL, add_body, 0)

        for b in range(NBUF):
            issue_gather(b, b)

        def step_body(s, _):
            for b in range(NBUF):
                p = s * NBUF + b
                wait_gather(b)

                @pl.when(s > 0)
                def _():
                    wait_out(b)

                add_pos(b)
                issue_out(p, b)

                @pl.when(p + NBUF < pairs)
                def _():
                    issue_gather(p + NBUF, b)
            return 0

        lax.fori_loop(0, steps, step_body, 0)
        for b in range(NBUF):
            wait_out(b)

    return k


def kernel(x, token_table, pos_table):
    x2 = x.reshape(FLAT // CHUNK, CHUNK).astype(jnp.int32)
    out = _sc_kernel()(x2, token_table, pos_table)
    return out.reshape(B, T, D)


# speedup vs baseline: 1.4924x; 1.0013x over previous
import functools

import jax
import jax.numpy as jnp
from jax import lax
from jax.experimental import pallas as pl
from jax.experimental.pallas import tpu as pltpu
from jax.experimental.pallas import tpu_sc as plsc

B = 4096
T = 200
D = 32
FLAT = B * T
HALVES = ((0, 104), (104, 96))
PAIR = 200
NBUF = 4
ROW_UNROLL = 8


@functools.lru_cache(maxsize=None)
def _sc_kernel():
    info = plsc.get_sparse_core_info()
    NC, NS = info.num_cores, info.num_subcores
    NW = NC * NS
    rows_per_w = FLAT // NW
    pairs = rows_per_w // PAIR
    steps = pairs // NBUF

    mesh = plsc.VectorSubcoreMesh(core_axis_name="c", subcore_axis_name="s")

    @functools.partial(
        pl.kernel,
        mesh=mesh,
        compiler_params=pltpu.CompilerParams(use_tc_tiling_on_sc=False),
        out_type=jax.ShapeDtypeStruct((B, T, D), jnp.float32),
        scratch_types=[
            pltpu.VMEM((pairs, T), jnp.int32),
            pltpu.VMEM((T, D), jnp.float32),
            pltpu.VMEM((NBUF, PAIR, D), jnp.float32),
            pltpu.VMEM((NBUF, PAIR, D), jnp.float32),
            *([pltpu.SemaphoreType.DMA] * (2 * NBUF)),
        ],
    )
    def k(x_hbm, tok_hbm, pos_hbm, out_hbm, idx_v, pos_v, gbuf, obuf, *sems):
        gsem = sems[:NBUF]
        osem = sems[NBUF:]
        cid = lax.axis_index("c")
        sid = lax.axis_index("s")
        wid = sid * NC + cid
        bq0 = wid * pairs

        pltpu.sync_copy(x_hbm.at[pl.ds(bq0, pairs), :], idx_v)
        pltpu.sync_copy(pos_hbm, pos_v)

        def issue_gather(p, b):
            for start, n in HALVES:
                pltpu.async_copy(
                    tok_hbm.at[idx_v.at[p, pl.ds(start, n)]],
                    gbuf.at[b, pl.ds(start, n), :], gsem[b])

        def wait_gather(b):
            for start, n in HALVES:
                pltpu.make_async_copy(
                    tok_hbm.at[idx_v.at[0, pl.ds(0, n)]],
                    gbuf.at[b, pl.ds(start, n), :], gsem[b]).wait()

        def issue_out(p, b):
            pltpu.async_copy(obuf.at[b], out_hbm.at[bq0 + p], osem[b])

        def wait_out(b):
            pltpu.make_async_copy(
                obuf.at[b], out_hbm.at[bq0], osem[b]).wait()

        def add_pos(b):
            def add_body(i, _):
                r0 = i * ROW_UNROLL
                for dr in range(ROW_UNROLL):
                    r = r0 + dr
                    for h in (0, 16):
                        obuf[b, r, pl.ds(h, 16)] = (
                            gbuf[b, r, pl.ds(h, 16)] + pos_v[r, pl.ds(h, 16)])
                return 0
            lax.fori_loop(0, T // ROW_UNROLL, add_body, 0)

        for b in range(NBUF):
            issue_gather(b, b)

        def step_body(s, _):
            for b in range(NBUF):
                p = s * NBUF + b
                wait_gather(b)

                @pl.when(s > 0)
                def _():
                    wait_out(b)

                add_pos(b)
                issue_out(p, b)

                @pl.when(p + NBUF < pairs)
                def _():
                    issue_gather(p + NBUF, b)
            return 0

        lax.fori_loop(0, steps, step_body, 0)
        for b in range(NBUF):
            wait_out(b)

    return k


def kernel(x, token_table, pos_table):
    return _sc_kernel()(x.astype(jnp.int32), token_table, pos_table)


# speedup vs baseline: 2.0323x; 1.3618x over previous
import functools

import jax
import jax.numpy as jnp
from jax import lax
from jax.experimental import pallas as pl
from jax.experimental.pallas import tpu as pltpu
from jax.experimental.pallas import tpu_sc as plsc

B = 4096
T = 200
D = 32
FLAT = B * T
HALVES = ((0, 104), (104, 96))
PAIR = 200
NBUF = 4
ROW_UNROLL = 8


@functools.lru_cache(maxsize=None)
def _sc_kernel():
    info = plsc.get_sparse_core_info()
    NC, NS = info.num_cores, info.num_subcores
    NW = NC * NS
    rows_per_w = FLAT // NW
    pairs = rows_per_w // PAIR
    steps = pairs // NBUF

    mesh = plsc.VectorSubcoreMesh(core_axis_name="c", subcore_axis_name="s")

    @functools.partial(
        pl.kernel,
        mesh=mesh,
        compiler_params=pltpu.CompilerParams(use_tc_tiling_on_sc=False),
        out_type=jax.ShapeDtypeStruct((B, T, 128), jnp.float32),
        scratch_types=[
            pltpu.VMEM((pairs, T), jnp.int32),
            pltpu.VMEM((T, D), jnp.float32),
            pltpu.VMEM((NBUF, PAIR, D), jnp.float32),
            pltpu.VMEM((NBUF, PAIR, D), jnp.float32),
            *([pltpu.SemaphoreType.DMA] * (2 * NBUF)),
        ],
    )
    def k(x_hbm, tok_hbm, pos_hbm, out_hbm, idx_v, pos_v, gbuf, obuf, *sems):
        gsem = sems[:NBUF]
        osem = sems[NBUF:]
        cid = lax.axis_index("c")
        sid = lax.axis_index("s")
        wid = sid * NC + cid
        bq0 = wid * pairs

        pltpu.sync_copy(x_hbm.at[pl.ds(bq0, pairs), :], idx_v)
        pltpu.sync_copy(pos_hbm, pos_v)

        def issue_gather(p, b):
            for start, n in HALVES:
                pltpu.async_copy(
                    tok_hbm.at[idx_v.at[p, pl.ds(start, n)]],
                    gbuf.at[b, pl.ds(start, n), :], gsem[b])

        def wait_gather(b):
            for start, n in HALVES:
                pltpu.make_async_copy(
                    tok_hbm.at[idx_v.at[0, pl.ds(0, n)]],
                    gbuf.at[b, pl.ds(start, n), :], gsem[b]).wait()

        def issue_out(p, b):
            pltpu.async_copy(
                obuf.at[b], out_hbm.at[bq0 + p, :, pl.ds(0, D)], osem[b])

        def wait_out(b):
            pltpu.make_async_copy(
                obuf.at[b], out_hbm.at[bq0, :, pl.ds(0, D)], osem[b]).wait()

        def add_pos(b):
            def add_body(i, _):
                r0 = i * ROW_UNROLL
                for dr in range(ROW_UNROLL):
                    r = r0 + dr
                    for h in (0, 16):
                        obuf[b, r, pl.ds(h, 16)] = (
                            gbuf[b, r, pl.ds(h, 16)] + pos_v[r, pl.ds(h, 16)])
                return 0
            lax.fori_loop(0, T // ROW_UNROLL, add_body, 0)

        for b in range(NBUF):
            issue_gather(b, b)

        def step_body(s, _):
            for b in range(NBUF):
                p = s * NBUF + b
                wait_gather(b)

                @pl.when(s > 0)
                def _():
                    wait_out(b)

                add_pos(b)
                issue_out(p, b)

                @pl.when(p + NBUF < pairs)
                def _():
                    issue_gather(p + NBUF, b)
            return 0

        lax.fori_loop(0, steps, step_body, 0)
        for b in range(NBUF):
            wait_out(b)

    return k


def kernel(x, token_table, pos_table):
    out = _sc_kernel()(x.astype(jnp.int32), token_table, pos_table)
    return out[:, :, :D]
